# baseline (speedup 1.0000x reference)
import functools

import jax
import jax.numpy as jnp
from jax import lax
from jax.experimental import pallas as pl
from jax.experimental.pallas import tpu as pltpu
from jax.experimental.pallas import tpu_sc as plsc

NC = 2
NS = 16
NW = NC * NS
LANES = 16


def _make_sc_kernel(n_nodes: int, d_feat: int, n_edges: int, chunk: int):
    assert n_edges % NW == 0
    e_per_w = n_edges // NW
    assert e_per_w % chunk == 0 and chunk % LANES == 0 and chunk % 8 == 0
    assert chunk <= 128
    n_steps = e_per_w // chunk
    n_groups = chunk // LANES
    assert n_steps % 2 == 1

    mesh = plsc.VectorSubcoreMesh(
        core_axis_name="c", subcore_axis_name="s",
        num_cores=NC, num_subcores=NS)

    @functools.partial(
        pl.kernel,
        out_type=jax.ShapeDtypeStruct((n_edges,), jnp.float32),
        mesh=mesh,
        compiler_params=pltpu.CompilerParams(needs_layout_passes=False),
        scratch_types=[
            pltpu.VMEM((e_per_w,), jnp.int32),
            pltpu.VMEM((e_per_w,), jnp.int32),
            pltpu.VMEM((2, chunk, d_feat), jnp.int32),
            pltpu.VMEM((2, chunk, d_feat), jnp.int32),
            pltpu.VMEM((e_per_w,), jnp.float32),
            pltpu.SemaphoreType.DMA,
            pltpu.SemaphoreType.DMA,
            pltpu.SemaphoreType.DMA,
            pltpu.SemaphoreType.DMA,
        ],
    )
    def sc_kernel(h_hbm, src_hbm, dst_hbm, out_hbm,
                  idx_s, idx_d, rows_s, rows_d, scores,
                  sem_s0, sem_s1, sem_d0, sem_d1):
        wid = lax.axis_index("s") * NC + lax.axis_index("c")
        lane = lax.broadcasted_iota(jnp.int32, (LANES,), 0)
        sem_s = (sem_s0, sem_s1)
        sem_d = (sem_d0, sem_d1)

        pltpu.sync_copy(src_hbm.at[pl.ds(wid * e_per_w, e_per_w)], idx_s)
        pltpu.sync_copy(dst_hbm.at[pl.ds(wid * e_per_w, e_per_w)], idx_d)

        def issue(step, b):
            pltpu.async_copy(
                h_hbm.at[idx_s.at[pl.ds(step * chunk, chunk)]],
                rows_s.at[b], sem_s[b])
            pltpu.async_copy(
                h_hbm.at[idx_d.at[pl.ds(step * chunk, chunk)]],
                rows_d.at[b], sem_d[b])

        def wait(b):
            pltpu.make_async_copy(
                h_hbm.at[idx_s.at[pl.ds(0, chunk)]], rows_s.at[b],
                sem_s[b]).wait()
            pltpu.make_async_copy(
                h_hbm.at[idx_d.at[pl.ds(0, chunk)]], rows_d.at[b],
                sem_d[b]).wait()

        w = 2 * LANES
        n_k = d_feat // w

        def compute(step, b):
            base = step * chunk

            @plsc.parallel_loop(0, chunk, step=1, unroll=4)
            def eloop(e):
                acc = None
                for k in range(n_k):
                    ps = plsc.bitcast(rows_s[b, e, pl.ds(k * LANES, LANES)],
                                      jnp.bfloat16)
                    pd = plsc.bitcast(rows_d[b, e, pl.ds(k * LANES, LANES)],
                                      jnp.bfloat16)
                    p = ps * pd
                    pa, pb = plsc.unpack(p, format=plsc.PackFormat.INTERLEAVED)
                    q = pa + pb
                    acc = q if acc is None else acc + q
                s = jnp.sum(acc)
                plsc.store_scatter(
                    scores, [jnp.full((LANES,), base + e, jnp.int32)],
                    jnp.broadcast_to(s, (LANES,)), mask=lane == 0)

        issue(0, 0)

        def pair(t, carry):
            s0 = 2 * t
            issue(s0 + 1, 1)
            wait(0)
            compute(s0, 0)
            issue(s0 + 2, 0)
            wait(1)
            compute(s0 + 1, 1)
            return carry

        lax.fori_loop(0, (n_steps - 1) // 2, pair, 0)
        wait(0)
        compute(n_steps - 1, 0)

        pltpu.sync_copy(scores, out_hbm.at[pl.ds(wid * e_per_w, e_per_w)])

    return sc_kernel


def kernel(h, edge_index):
    n_nodes, d_feat = h.shape
    n_edges = edge_index.shape[1]
    ei = edge_index.astype(jnp.int32)
    sc = _make_sc_kernel(n_nodes, d_feat, n_edges, chunk=80)
    h_packed = jax.lax.bitcast_convert_type(
        h.astype(jnp.bfloat16).reshape(n_nodes, d_feat // 2, 2), jnp.int32)
    h_padded = jnp.pad(h_packed, ((0, 0), (0, d_feat - d_feat // 2)))
    return sc(h_padded, ei[0], ei[1])


# speedup vs baseline: 1.1776x; 1.1776x over previous
import functools

import jax
import jax.numpy as jnp
from jax import lax
from jax.experimental import pallas as pl
from jax.experimental.pallas import tpu as pltpu
from jax.experimental.pallas import tpu_sc as plsc

NC = 2
NS = 16
NW = NC * NS
LANES = 16


def _make_sc_kernel(n_nodes: int, d_feat: int, n_edges: int, chunk: int):
    assert n_edges % NW == 0
    e_per_w = n_edges // NW
    assert e_per_w % chunk == 0 and chunk % LANES == 0 and chunk % 8 == 0
    assert chunk <= 128
    n_steps = e_per_w // chunk
    assert n_steps % 2 == 1

    mesh = plsc.VectorSubcoreMesh(
        core_axis_name="c", subcore_axis_name="s",
        num_cores=NC, num_subcores=NS)

    @functools.partial(
        pl.kernel,
        out_type=jax.ShapeDtypeStruct((n_edges,), jnp.float32),
        mesh=mesh,
        compiler_params=pltpu.CompilerParams(needs_layout_passes=False),
        scratch_types=[
            pltpu.VMEM((e_per_w,), jnp.int32),
            pltpu.VMEM((e_per_w,), jnp.int32),
            pltpu.VMEM((2, chunk, d_feat), jnp.float32),
            pltpu.VMEM((2, chunk, d_feat), jnp.float32),
            pltpu.VMEM((e_per_w,), jnp.float32),
            pltpu.SemaphoreType.DMA,
            pltpu.SemaphoreType.DMA,
            pltpu.SemaphoreType.DMA,
            pltpu.SemaphoreType.DMA,
        ],
    )
    def sc_kernel(h_hbm, src_hbm, dst_hbm, out_hbm,
                  idx_s, idx_d, rows_s, rows_d, scores,
                  sem_s0, sem_s1, sem_d0, sem_d1):
        wid = lax.axis_index("s") * NC + lax.axis_index("c")
        lane = lax.broadcasted_iota(jnp.int32, (LANES,), 0)
        sem_s = (sem_s0, sem_s1)
        sem_d = (sem_d0, sem_d1)

        pltpu.sync_copy(src_hbm.at[pl.ds(wid * e_per_w, e_per_w)], idx_s)
        pltpu.sync_copy(dst_hbm.at[pl.ds(wid * e_per_w, e_per_w)], idx_d)

        def issue(step, b):
            pltpu.async_copy(
                h_hbm.at[idx_s.at[pl.ds(step * chunk, chunk)]],
                rows_s.at[b], sem_s[b])
            pltpu.async_copy(
                h_hbm.at[idx_d.at[pl.ds(step * chunk, chunk)]],
                rows_d.at[b], sem_d[b])

        def wait(b):
            pltpu.make_async_copy(
                h_hbm.at[idx_s.at[pl.ds(0, chunk)]], rows_s.at[b],
                sem_s[b]).wait()
            pltpu.make_async_copy(
                h_hbm.at[idx_d.at[pl.ds(0, chunk)]], rows_d.at[b],
                sem_d[b]).wait()

        n_k = d_feat // LANES

        def compute(step, b):
            base = step * chunk

            @plsc.parallel_loop(0, chunk, step=1, unroll=8)
            def eloop(e):
                a0 = rows_s[b, e, pl.ds(0, LANES)] * rows_d[b, e, pl.ds(0, LANES)]
                a1 = (rows_s[b, e, pl.ds(LANES, LANES)]
                      * rows_d[b, e, pl.ds(LANES, LANES)])
                for k in range(2, n_k, 2):
                    a0 = a0 + (rows_s[b, e, pl.ds(k * LANES, LANES)]
                               * rows_d[b, e, pl.ds(k * LANES, LANES)])
                    a1 = a1 + (rows_s[b, e, pl.ds((k + 1) * LANES, LANES)]
                               * rows_d[b, e, pl.ds((k + 1) * LANES, LANES)])
                s = jnp.sum(a0 + a1)
                plsc.store_scatter(
                    scores, [jnp.full((LANES,), base + e, jnp.int32)],
                    jnp.broadcast_to(s, (LANES,)), mask=lane == 0)

        issue(0, 0)

        def pair(t, carry):
            s0 = 2 * t
            issue(s0 + 1, 1)
            wait(0)
            compute(s0, 0)
            issue(s0 + 2, 0)
            wait(1)
            compute(s0 + 1, 1)
            return carry

        lax.fori_loop(0, (n_steps - 1) // 2, pair, 0)
        wait(0)
        compute(n_steps - 1, 0)

        pltpu.sync_copy(scores, out_hbm.at[pl.ds(wid * e_per_w, e_per_w)])

    return sc_kernel


def kernel(h, edge_index):
    n_nodes, d_feat = h.shape
    n_edges = edge_index.shape[1]
    ei = edge_index.astype(jnp.int32)
    sc = _make_sc_kernel(n_nodes, d_feat, n_edges, chunk=80)
    return sc(h, ei[0], ei[1])


# speedup vs baseline: 1.1909x; 1.0113x over previous
import functools

import jax
import jax.numpy as jnp
from jax import lax
from jax.experimental import pallas as pl
from jax.experimental.pallas import tpu as pltpu
from jax.experimental.pallas import tpu_sc as plsc

NC = 2
NS = 16
NW = NC * NS
LANES = 16


def _make_sc_kernel(n_nodes: int, d_feat: int, n_edges: int, chunk: int):
    assert n_edges % NW == 0
    e_per_w = n_edges // NW
    assert e_per_w % chunk == 0 and chunk % LANES == 0 and chunk % 8 == 0
    assert chunk <= 128
    n_steps = e_per_w // chunk
    assert n_steps % 2 == 1

    mesh = plsc.VectorSubcoreMesh(
        core_axis_name="c", subcore_axis_name="s",
        num_cores=NC, num_subcores=NS)

    @functools.partial(
        pl.kernel,
        out_type=jax.ShapeDtypeStruct((n_edges,), jnp.float32),
        mesh=mesh,
        compiler_params=pltpu.CompilerParams(needs_layout_passes=False),
        scratch_types=[
            pltpu.VMEM((e_per_w,), jnp.int32),
            pltpu.VMEM((e_per_w,), jnp.int32),
            pltpu.VMEM((2, chunk, d_feat), jnp.float32),
            pltpu.VMEM((2, chunk, d_feat), jnp.float32),
            pltpu.VMEM((e_per_w,), jnp.float32),
            pltpu.SemaphoreType.DMA,
            pltpu.SemaphoreType.DMA,
            pltpu.SemaphoreType.DMA,
            pltpu.SemaphoreType.DMA,
        ],
    )
    def sc_kernel(h_hbm, src_hbm, dst_hbm, out_hbm,
                  idx_s, idx_d, rows_s, rows_d, scores,
                  sem_s0, sem_s1, sem_d0, sem_d1):
        wid = lax.axis_index("s") * NC + lax.axis_index("c")
        lane = lax.broadcasted_iota(jnp.int32, (LANES,), 0)
        sem_s = (sem_s0, sem_s1)
        sem_d = (sem_d0, sem_d1)

        pltpu.sync_copy(src_hbm.at[pl.ds(wid * e_per_w, e_per_w)], idx_s)
        pltpu.sync_copy(dst_hbm.at[pl.ds(wid * e_per_w, e_per_w)], idx_d)

        def issue(step, b):
            pltpu.async_copy(
                h_hbm.at[idx_s.at[pl.ds(step * chunk, chunk)]],
                rows_s.at[b], sem_s[b])
            pltpu.async_copy(
                h_hbm.at[idx_d.at[pl.ds(step * chunk, chunk)]],
                rows_d.at[b], sem_d[b])

        def wait(b):
            pltpu.make_async_copy(
                h_hbm.at[idx_s.at[pl.ds(0, chunk)]], rows_s.at[b],
                sem_s[b]).wait()
            pltpu.make_async_copy(
                h_hbm.at[idx_d.at[pl.ds(0, chunk)]], rows_d.at[b],
                sem_d[b]).wait()

        n_k = d_feat // LANES

        def compute(step, b):
            base = step * chunk

            @plsc.parallel_loop(0, chunk, step=1, unroll=4)
            def eloop(e):
                a0 = rows_s[b, e, pl.ds(0, LANES)] * rows_d[b, e, pl.ds(0, LANES)]
                a1 = (rows_s[b, e, pl.ds(LANES, LANES)]
                      * rows_d[b, e, pl.ds(LANES, LANES)])
                for k in range(2, n_k, 2):
                    a0 = a0 + (rows_s[b, e, pl.ds(k * LANES, LANES)]
                               * rows_d[b, e, pl.ds(k * LANES, LANES)])
                    a1 = a1 + (rows_s[b, e, pl.ds((k + 1) * LANES, LANES)]
                               * rows_d[b, e, pl.ds((k + 1) * LANES, LANES)])
                s = jnp.sum(a0 + a1)
                plsc.store_scatter(
                    scores, [jnp.full((LANES,), base + e, jnp.int32)],
                    jnp.broadcast_to(s, (LANES,)), mask=lane == 0)

        issue(0, 0)

        def pair(t, carry):
            s0 = 2 * t
            issue(s0 + 1, 1)
            wait(0)
            compute(s0, 0)
            issue(s0 + 2, 0)
            wait(1)
            compute(s0 + 1, 1)
            return carry

        lax.fori_loop(0, (n_steps - 1) // 2, pair, 0)
        wait(0)
        compute(n_steps - 1, 0)

        pltpu.sync_copy(scores, out_hbm.at[pl.ds(wid * e_per_w, e_per_w)])

    return sc_kernel


def kernel(h, edge_index):
    n_nodes, d_feat = h.shape
    n_edges = edge_index.shape[1]
    ei = edge_index.astype(jnp.int32)
    sc = _make_sc_kernel(n_nodes, d_feat, n_edges, chunk=80)
    return sc(h, ei[0], ei[1])


# speedup vs baseline: 1.2624x; 1.0600x over previous
import functools

import jax
import jax.numpy as jnp
from jax import lax
from jax.experimental import pallas as pl
from jax.experimental.pallas import tpu as pltpu
from jax.experimental.pallas import tpu_sc as plsc

NC = 2
NS = 16
NW = NC * NS
LANES = 16


def _make_sc_kernel(n_nodes: int, d_feat: int, n_edges: int, chunk: int):
    assert n_edges % NW == 0
    e_per_w = n_edges // NW
    assert chunk % LANES == 0 and chunk % 8 == 0
    assert chunk <= 128
    n_full = e_per_w // chunk
    tail = e_per_w - n_full * chunk
    assert n_full % 2 == 0 and 0 < tail <= chunk and tail % 8 == 0

    mesh = plsc.VectorSubcoreMesh(
        core_axis_name="c", subcore_axis_name="s",
        num_cores=NC, num_subcores=NS)

    @functools.partial(
        pl.kernel,
        out_type=jax.ShapeDtypeStruct((n_edges,), jnp.float32),
        mesh=mesh,
        compiler_params=pltpu.CompilerParams(needs_layout_passes=False),
        scratch_types=[
            pltpu.VMEM((e_per_w,), jnp.int32),
            pltpu.VMEM((e_per_w,), jnp.int32),
            pltpu.VMEM((2, chunk, d_feat), jnp.float32),
            pltpu.VMEM((2, chunk, d_feat), jnp.float32),
            pltpu.VMEM((e_per_w,), jnp.float32),
            pltpu.SemaphoreType.DMA,
            pltpu.SemaphoreType.DMA,
            pltpu.SemaphoreType.DMA,
            pltpu.SemaphoreType.DMA,
        ],
    )
    def sc_kernel(h_hbm, src_hbm, dst_hbm, out_hbm,
                  idx_s, idx_d, rows_s, rows_d, scores,
                  sem_s0, sem_s1, sem_d0, sem_d1):
        wid = lax.axis_index("s") * NC + lax.axis_index("c")
        lane = lax.broadcasted_iota(jnp.int32, (LANES,), 0)
        sem_s = (sem_s0, sem_s1)
        sem_d = (sem_d0, sem_d1)

        pltpu.sync_copy(src_hbm.at[pl.ds(wid * e_per_w, e_per_w)], idx_s)
        pltpu.sync_copy(dst_hbm.at[pl.ds(wid * e_per_w, e_per_w)], idx_d)

        def issue(step, b):
            pltpu.async_copy(
                h_hbm.at[idx_s.at[pl.ds(step * chunk, chunk)]],
                rows_s.at[b], sem_s[b])
            pltpu.async_copy(
                h_hbm.at[idx_d.at[pl.ds(step * chunk, chunk)]],
                rows_d.at[b], sem_d[b])

        def wait(b):
            pltpu.make_async_copy(
                h_hbm.at[idx_s.at[pl.ds(0, chunk)]], rows_s.at[b],
                sem_s[b]).wait()
            pltpu.make_async_copy(
                h_hbm.at[idx_d.at[pl.ds(0, chunk)]], rows_d.at[b],
                sem_d[b]).wait()

        def issue_tail(b):
            pltpu.async_copy(
                h_hbm.at[idx_s.at[pl.ds(n_full * chunk, tail)]],
                rows_s.at[b].at[pl.ds(0, tail)], sem_s[b])
            pltpu.async_copy(
                h_hbm.at[idx_d.at[pl.ds(n_full * chunk, tail)]],
                rows_d.at[b].at[pl.ds(0, tail)], sem_d[b])

        def wait_tail(b):
            pltpu.make_async_copy(
                h_hbm.at[idx_s.at[pl.ds(0, tail)]],
                rows_s.at[b].at[pl.ds(0, tail)], sem_s[b]).wait()
            pltpu.make_async_copy(
                h_hbm.at[idx_d.at[pl.ds(0, tail)]],
                rows_d.at[b].at[pl.ds(0, tail)], sem_d[b]).wait()

        n_k = d_feat // LANES

        def compute(step, b, width=None):
            base = step * chunk

            @plsc.parallel_loop(0, width or chunk, step=1, unroll=4)
            def eloop(e):
                a0 = rows_s[b, e, pl.ds(0, LANES)] * rows_d[b, e, pl.ds(0, LANES)]
                a1 = (rows_s[b, e, pl.ds(LANES, LANES)]
                      * rows_d[b, e, pl.ds(LANES, LANES)])
                for k in range(2, n_k, 2):
                    a0 = a0 + (rows_s[b, e, pl.ds(k * LANES, LANES)]
                               * rows_d[b, e, pl.ds(k * LANES, LANES)])
                    a1 = a1 + (rows_s[b, e, pl.ds((k + 1) * LANES, LANES)]
                               * rows_d[b, e, pl.ds((k + 1) * LANES, LANES)])
                s = jnp.sum(a0 + a1)
                plsc.store_scatter(
                    scores, [jnp.full((LANES,), base + e, jnp.int32)],
                    jnp.broadcast_to(s, (LANES,)), mask=lane == 0)

        issue(0, 0)

        def pair(t, carry):
            s0 = 2 * t
            issue(s0 + 1, 1)
            wait(0)
            compute(s0, 0)

            @pl.when(s0 + 2 < n_full)
            def _issue_next():
                issue(s0 + 2, 0)

            @pl.when(s0 + 2 == n_full)
            def _issue_tail():
                issue_tail(0)

            wait(1)
            compute(s0 + 1, 1)
            return carry

        lax.fori_loop(0, n_full // 2, pair, 0)
        wait_tail(0)
        compute(n_full, 0, width=tail)

        pltpu.sync_copy(scores, out_hbm.at[pl.ds(wid * e_per_w, e_per_w)])

    return sc_kernel


def kernel(h, edge_index):
    n_nodes, d_feat = h.shape
    n_edges = edge_index.shape[1]
    ei = edge_index.astype(jnp.int32)
    sc = _make_sc_kernel(n_nodes, d_feat, n_edges, chunk=128)
    return sc(h, ei[0], ei[1])
